# baseline (speedup 1.0000x reference)
import functools

import jax
import jax.numpy as jnp
from jax import lax
from jax.experimental import pallas as pl
from jax.experimental.pallas import tpu as pltpu
from jax.experimental.pallas import tpu_sc as plsc

EPS_BN = 1e-5
NT = 512
CH = 16


def _k1_body(xyz1_ref, xyz2_ref, idx_ref, w_ref):
    S = xyz2_ref.shape[-1]
    nt = xyz1_ref.shape[-1]
    x1 = xyz1_ref[0]
    x2 = xyz2_ref[0]
    m = jax.lax.dot_general(x1, x2, (((0,), (0,)), ((), ())),
                            preferred_element_type=jnp.float32)
    n1 = (x1[0] * x1[0] + x1[1] * x1[1]) + x1[2] * x1[2]
    n2 = (x2[0] * x2[0] + x2[1] * x2[1]) + x2[2] * x2[2]
    d = (-2.0 * m + n1[:, None]) + n2[None, :]

    iota = jax.lax.broadcasted_iota(jnp.int32, (nt, S), 1)
    idxs = []
    vals = []
    for _ in range(3):
        mn = jnp.min(d, axis=1)
        ii = jnp.min(jnp.where(d == mn[:, None], iota, S), axis=1)
        idxs.append(ii)
        vals.append(mn)
        d = jnp.where(iota == ii[:, None], jnp.float32(jnp.inf), d)

    r0 = 1.0 / (vals[0] + 1e-8)
    r1 = 1.0 / (vals[1] + 1e-8)
    r2 = 1.0 / (vals[2] + 1e-8)
    rs = r0 + r1 + r2

    b = pl.program_id(0)
    base = b * S
    idx_ref[0] = jnp.stack([idxs[0] + base, idxs[1] + base, idxs[2] + base])
    w_ref[0] = jnp.stack([r0 / rs, r1 / rs, r2 / rs])


def _sc_interp_body(table, ii, we, out,
                    iv, ra, rb, wa, wb, ob,
                    sg0, sg1):
    Q = out.shape[0]
    C2 = out.shape[1]
    NW = 32
    qpw = Q // NW
    nchunks = qpw // CH
    wid = lax.axis_index("s") * 2 + lax.axis_index("c")
    base = pl.multiple_of(wid * qpw, 8)

    pltpu.sync_copy(ii.at[pl.ds(base * 3, qpw * 3)], iv)

    rbufs = (ra, rb)
    wbufs = (wa, wb)
    sgs = (sg0, sg1)

    def issue(g, buf):
        off = pl.multiple_of(g * CH * 3, 8)
        qs = pl.multiple_of(base + g * CH, 8)
        pltpu.async_copy(table.at[iv.at[pl.ds(off, CH * 3)]],
                         rbufs[buf], sgs[buf])
        pltpu.async_copy(we.at[pl.ds(qs, CH)], wbufs[buf], sgs[buf])

    def drain(g, buf):
        off = pl.multiple_of(g * CH * 3, 8)
        qs = pl.multiple_of(base + g * CH, 8)
        pltpu.make_async_copy(table.at[iv.at[pl.ds(off, CH * 3)]],
                              rbufs[buf], sgs[buf]).wait()
        pltpu.make_async_copy(we.at[pl.ds(qs, CH)],
                              wbufs[buf], sgs[buf]).wait()

    issue(0, 0)
    issue(1, 1)

    def step(g, buf):
        drain(g, buf)
        rr = rbufs[buf]
        ww = wbufs[buf]
        for q in range(CH):
            a0 = ww[q, 0]
            a1 = ww[q, 1]
            a2 = ww[q, 2]
            for c in range(C2 // 16):
                sl = pl.ds(c * 16, 16)
                ob[q, sl] = ((a0 * rr[3 * q, sl] + a1 * rr[3 * q + 1, sl])
                             + a2 * rr[3 * q + 2, sl])

        @pl.when(g + 2 < nchunks)
        def _():
            issue(g + 2, buf)

        pltpu.sync_copy(ob, out.at[pl.ds(base + g * CH, CH)])

    def body(gg, carry):
        step(gg * 2, 0)
        step(gg * 2 + 1, 1)
        return carry

    lax.fori_loop(0, nchunks // 2, body, 0)


def _k2_body(f1_ref, interp_ref, w1a_ref, w1b_ref, b1_ref, y1_ref, stats_ref):
    f1 = f1_ref[0]
    itp = interp_ref[0]
    y = jax.lax.dot_general(w1a_ref[...], f1, (((1,), (0,)), ((), ())),
                            preferred_element_type=jnp.float32)
    y = y + jax.lax.dot_general(w1b_ref[...], itp, (((1,), (1,)), ((), ())),
                                preferred_element_type=jnp.float32)
    y = y + b1_ref[...]
    y1_ref[0] = y

    b = pl.program_id(0)
    n = pl.program_id(1)

    @pl.when(jnp.logical_and(b == 0, n == 0))
    def _():
        stats_ref[...] = jnp.zeros_like(stats_ref)

    s = jnp.sum(y, axis=1)
    q = jnp.sum(y * y, axis=1)
    stats_ref[...] += jnp.stack([s, q], axis=0)


def _k3_body(y1_ref, scale_ref, shift_ref, w2_ref, b2_ref, y2_ref, stats_ref):
    x = y1_ref[0]
    x = jnp.maximum(x * scale_ref[...] + shift_ref[...], 0.0)
    y = jax.lax.dot_general(w2_ref[...], x, (((1,), (0,)), ((), ())),
                            preferred_element_type=jnp.float32)
    y = y + b2_ref[...]
    y2_ref[0] = y

    b = pl.program_id(0)
    n = pl.program_id(1)

    @pl.when(jnp.logical_and(b == 0, n == 0))
    def _():
        stats_ref[...] = jnp.zeros_like(stats_ref)

    s = jnp.sum(y, axis=1)
    q = jnp.sum(y * y, axis=1)
    stats_ref[...] += jnp.stack([s, q], axis=0)


def _k4_body(y2_ref, scale_ref, shift_ref, out_ref):
    out_ref[0] = jnp.maximum(y2_ref[0] * scale_ref[...] + shift_ref[...], 0.0)


def kernel(point_coordinates1, point_coordinates2, features1, features2,
           W1, b1, g1, be1, W2, b2, g2, be2):
    B, _, N = point_coordinates1.shape
    S = point_coordinates2.shape[-1]
    C1 = features1.shape[1]
    C2 = features2.shape[1]
    CM1 = W1.shape[0]
    CM2 = W2.shape[0]
    nb = N // NT
    Q = B * N
    qpw = Q // 32

    gidx, wgt = pl.pallas_call(
        _k1_body,
        grid=(B, nb),
        in_specs=[
            pl.BlockSpec((1, 3, NT), lambda b, n: (b, 0, n)),
            pl.BlockSpec((1, 3, S), lambda b, n: (b, 0, 0)),
        ],
        out_specs=[
            pl.BlockSpec((1, 3, NT), lambda b, n: (b, 0, n)),
            pl.BlockSpec((1, 3, NT), lambda b, n: (b, 0, n)),
        ],
        out_shape=[
            jax.ShapeDtypeStruct((B, 3, N), jnp.int32),
            jax.ShapeDtypeStruct((B, 3, N), jnp.float32),
        ],
        compiler_params=pltpu.CompilerParams(
            dimension_semantics=("arbitrary", "arbitrary")),
    )(point_coordinates1, point_coordinates2)

    table = features2.transpose(0, 2, 1).reshape(B * S, C2)
    i0 = gidx[:, 0, :].reshape(Q)
    i1 = gidx[:, 1, :].reshape(Q)
    i2 = gidx[:, 2, :].reshape(Q)
    w0 = wgt[:, 0, :].reshape(Q)
    w1 = wgt[:, 1, :].reshape(Q)
    w2 = wgt[:, 2, :].reshape(Q)

    mesh = plsc.VectorSubcoreMesh(core_axis_name="c", subcore_axis_name="s")
    interp = pl.kernel(
        _sc_interp_body,
        mesh=mesh,
        out_type=jax.ShapeDtypeStruct((Q, C2), jnp.float32),
        scratch_types=[
            pltpu.VMEM((3 * qpw,), jnp.int32),
            pltpu.VMEM((3 * CH, C2), jnp.float32),
            pltpu.VMEM((3 * CH, C2), jnp.float32),
            pltpu.VMEM((CH, 3, 16), jnp.float32),
            pltpu.VMEM((CH, 3, 16), jnp.float32),
            pltpu.VMEM((CH, C2), jnp.float32),
            pltpu.SemaphoreType.DMA,
            pltpu.SemaphoreType.DMA,
        ],
    )(table,
      jnp.stack([i0, i1, i2], axis=1).reshape(3 * Q),
      jnp.broadcast_to(jnp.stack([w0, w1, w2], axis=1)[:, :, None],
                       (Q, 3, 16)))

    interp3 = interp.reshape(B, N, C2)

    y1, stats1 = pl.pallas_call(
        _k2_body,
        grid=(B, nb),
        in_specs=[
            pl.BlockSpec((1, C1, NT), lambda b, n: (b, 0, n)),
            pl.BlockSpec((1, NT, C2), lambda b, n: (b, n, 0)),
            pl.BlockSpec((CM1, C1), lambda b, n: (0, 0)),
            pl.BlockSpec((CM1, C2), lambda b, n: (0, 0)),
            pl.BlockSpec((CM1, 1), lambda b, n: (0, 0)),
        ],
        out_specs=[
            pl.BlockSpec((1, CM1, NT), lambda b, n: (b, 0, n)),
            pl.BlockSpec((2, CM1), lambda b, n: (0, 0)),
        ],
        out_shape=[
            jax.ShapeDtypeStruct((B, CM1, N), jnp.float32),
            jax.ShapeDtypeStruct((2, CM1), jnp.float32),
        ],
        compiler_params=pltpu.CompilerParams(
            dimension_semantics=("arbitrary", "arbitrary")),
    )(features1, interp3, W1[:, :C1], W1[:, C1:], b1[:, None])

    mean1 = stats1[0] / Q
    var1 = stats1[1] / Q - mean1 * mean1
    sc1 = g1 / jnp.sqrt(var1 + EPS_BN)
    sh1 = be1 - mean1 * sc1

    y2, stats2 = pl.pallas_call(
        _k3_body,
        grid=(B, nb),
        in_specs=[
            pl.BlockSpec((1, CM1, NT), lambda b, n: (b, 0, n)),
            pl.BlockSpec((CM1, 1), lambda b, n: (0, 0)),
            pl.BlockSpec((CM1, 1), lambda b, n: (0, 0)),
            pl.BlockSpec((CM2, CM1), lambda b, n: (0, 0)),
            pl.BlockSpec((CM2, 1), lambda b, n: (0, 0)),
        ],
        out_specs=[
            pl.BlockSpec((1, CM2, NT), lambda b, n: (b, 0, n)),
            pl.BlockSpec((2, CM2), lambda b, n: (0, 0)),
        ],
        out_shape=[
            jax.ShapeDtypeStruct((B, CM2, N), jnp.float32),
            jax.ShapeDtypeStruct((2, CM2), jnp.float32),
        ],
        compiler_params=pltpu.CompilerParams(
            dimension_semantics=("arbitrary", "arbitrary")),
    )(y1, sc1[:, None], sh1[:, None], W2, b2[:, None])

    mean2 = stats2[0] / Q
    var2 = stats2[1] / Q - mean2 * mean2
    sc2 = g2 / jnp.sqrt(var2 + EPS_BN)
    sh2 = be2 - mean2 * sc2

    out = pl.pallas_call(
        _k4_body,
        grid=(B,),
        in_specs=[
            pl.BlockSpec((1, CM2, N), lambda b: (b, 0, 0)),
            pl.BlockSpec((CM2, 1), lambda b: (0, 0)),
            pl.BlockSpec((CM2, 1), lambda b: (0, 0)),
        ],
        out_specs=pl.BlockSpec((1, CM2, N), lambda b: (b, 0, 0)),
        out_shape=jax.ShapeDtypeStruct((B, CM2, N), jnp.float32),
    )(y2, sc2[:, None], sh2[:, None])

    return out


# speedup vs baseline: 1.5085x; 1.5085x over previous
import functools

import jax
import jax.numpy as jnp
from jax import lax
from jax.experimental import pallas as pl
from jax.experimental.pallas import tpu as pltpu
from jax.experimental.pallas import tpu_sc as plsc

EPS_BN = 1e-5
NT = 512
CH = 16


def _top3(x1, x2, nt, S):
    m = jax.lax.dot_general(x1, x2, (((0,), (0,)), ((), ())),
                            preferred_element_type=jnp.float32)
    n1 = (x1[0] * x1[0] + x1[1] * x1[1]) + x1[2] * x1[2]
    n2 = (x2[0] * x2[0] + x2[1] * x2[1]) + x2[2] * x2[2]
    d = (-2.0 * m + n1[:, None]) + n2[None, :]

    iota = jax.lax.broadcasted_iota(jnp.int32, (nt, S), 1)
    idxs = []
    vals = []
    for _ in range(3):
        mn = jnp.min(d, axis=1)
        ii = jnp.min(jnp.where(d == mn[:, None], iota, S), axis=1)
        idxs.append(ii)
        vals.append(mn)
        d = jnp.where(iota == ii[:, None], jnp.float32(jnp.inf), d)

    r0 = 1.0 / (vals[0] + 1e-8)
    r1 = 1.0 / (vals[1] + 1e-8)
    r2 = 1.0 / (vals[2] + 1e-8)
    rs = r0 + r1 + r2
    return idxs, (r0 / rs, r1 / rs, r2 / rs)


def _k1a_body(xyz1_ref, xyz2_ref, idx_ref, w_ref):
    S = xyz2_ref.shape[-1]
    nt = xyz1_ref.shape[-1]
    idxs, ws = _top3(xyz1_ref[0], xyz2_ref[0], nt, S)
    base = pl.program_id(0) * S
    idx_ref[0] = jnp.stack([idxs[0] + base, idxs[1] + base, idxs[2] + base])
    w_ref[0] = jnp.stack(list(ws))


def _k1b_body(xyz1_ref, xyz2_ref, f1_ref, f2_ref, w1a_ref, w1b_ref, b1_ref,
              y1_ref, stats_ref):
    S = xyz2_ref.shape[-1]
    nt = xyz1_ref.shape[-1]
    idxs, ws = _top3(xyz1_ref[0], xyz2_ref[0], nt, S)

    iota_s = jax.lax.broadcasted_iota(jnp.int32, (S, nt), 0)
    at = jnp.where(iota_s == idxs[0][None, :], ws[0][None, :], 0.0)
    at = at + jnp.where(iota_s == idxs[1][None, :], ws[1][None, :], 0.0)
    at = at + jnp.where(iota_s == idxs[2][None, :], ws[2][None, :], 0.0)

    interp = jax.lax.dot_general(f2_ref[0], at, (((1,), (0,)), ((), ())),
                                 preferred_element_type=jnp.float32,
                                 precision=jax.lax.Precision.HIGHEST)

    y = jax.lax.dot_general(w1a_ref[...], f1_ref[0], (((1,), (0,)), ((), ())),
                            preferred_element_type=jnp.float32)
    y = y + jax.lax.dot_general(w1b_ref[...], interp, (((1,), (0,)), ((), ())),
                                preferred_element_type=jnp.float32)
    y = y + b1_ref[...]
    y1_ref[0] = y

    b = pl.program_id(0)
    n = pl.program_id(1)

    @pl.when(jnp.logical_and(b == 0, n == 0))
    def _():
        stats_ref[...] = jnp.zeros_like(stats_ref)

    s = jnp.sum(y, axis=1)
    q = jnp.sum(y * y, axis=1)
    stats_ref[...] += jnp.stack([s, q], axis=0)


def _sc_interp_body(table, i0, i1, i2, w0e, w1e, w2e, out,
                    iv0, iv1, iv2,
                    r0a, r0b, r1a, r1b, r2a, r2b,
                    wb0a, wb0b, wb1a, wb1b, wb2a, wb2b, ob,
                    sg0, sg1):
    Q = out.shape[0]
    C2 = out.shape[1]
    NW = 32
    qpw = Q // NW
    nchunks = qpw // CH
    wid = lax.axis_index("s") * 2 + lax.axis_index("c")
    base = pl.multiple_of(wid * qpw, 8)

    pltpu.sync_copy(i0.at[pl.ds(base, qpw)], iv0)
    pltpu.sync_copy(i1.at[pl.ds(base, qpw)], iv1)
    pltpu.sync_copy(i2.at[pl.ds(base, qpw)], iv2)

    rbufs = ((r0a, r1a, r2a), (r0b, r1b, r2b))
    wbufs = ((wb0a, wb1a, wb2a), (wb0b, wb1b, wb2b))
    ivs = (iv0, iv1, iv2)
    wes = (w0e, w1e, w2e)
    sgs = (sg0, sg1)

    def issue(g, buf):
        off = pl.multiple_of(g * CH, 8)
        qs = pl.multiple_of(base + g * CH, 8)
        for k in range(3):
            pltpu.async_copy(table.at[ivs[k].at[pl.ds(off, CH)]],
                             rbufs[buf][k], sgs[buf])
            pltpu.async_copy(wes[k].at[pl.ds(qs, CH)], wbufs[buf][k], sgs[buf])

    def drain(g, buf):
        off = pl.multiple_of(g * CH, 8)
        qs = pl.multiple_of(base + g * CH, 8)
        for k in range(3):
            pltpu.make_async_copy(table.at[ivs[k].at[pl.ds(off, CH)]],
                                  rbufs[buf][k], sgs[buf]).wait()
            pltpu.make_async_copy(wes[k].at[pl.ds(qs, CH)],
                                  wbufs[buf][k], sgs[buf]).wait()

    issue(0, 0)
    issue(1, 1)

    def step(g, buf):
        drain(g, buf)
        rb0, rb1, rb2 = rbufs[buf]
        wb0, wb1, wb2 = wbufs[buf]
        for q in range(CH):
            a0 = wb0[q]
            a1 = wb1[q]
            a2 = wb2[q]
            for c in range(C2 // 16):
                sl = pl.ds(c * 16, 16)
                ob[q, sl] = (a0 * rb0[q, sl] + a1 * rb1[q, sl]) + a2 * rb2[q, sl]

        @pl.when(g + 2 < nchunks)
        def _():
            issue(g + 2, buf)

        pltpu.sync_copy(ob, out.at[pl.ds(base + g * CH, CH)])

    def body(gg, carry):
        step(gg * 2, 0)
        step(gg * 2 + 1, 1)
        return carry

    lax.fori_loop(0, nchunks // 2, body, 0)


def _k2_body(f1_ref, interp_ref, w1a_ref, w1b_ref, b1_ref, y1_ref, stats_ref):
    y = jax.lax.dot_general(w1a_ref[...], f1_ref[0], (((1,), (0,)), ((), ())),
                            preferred_element_type=jnp.float32)
    y = y + jax.lax.dot_general(w1b_ref[...], interp_ref[0],
                                (((1,), (1,)), ((), ())),
                                preferred_element_type=jnp.float32)
    y = y + b1_ref[...]
    y1_ref[0] = y

    b = pl.program_id(0)
    n = pl.program_id(1)

    @pl.when(jnp.logical_and(b == 0, n == 0))
    def _():
        stats_ref[...] = jnp.zeros_like(stats_ref)

    s = jnp.sum(y, axis=1)
    q = jnp.sum(y * y, axis=1)
    stats_ref[...] += jnp.stack([s, q], axis=0)


def _k3_body(y1_ref, scale_ref, shift_ref, w2_ref, b2_ref, y2_ref, stats_ref):
    x = y1_ref[0]
    x = jnp.maximum(x * scale_ref[...] + shift_ref[...], 0.0)
    y = jax.lax.dot_general(w2_ref[...], x, (((1,), (0,)), ((), ())),
                            preferred_element_type=jnp.float32)
    y = y + b2_ref[...]
    y2_ref[0] = y

    b = pl.program_id(0)
    n = pl.program_id(1)

    @pl.when(jnp.logical_and(b == 0, n == 0))
    def _():
        stats_ref[...] = jnp.zeros_like(stats_ref)

    s = jnp.sum(y, axis=1)
    q = jnp.sum(y * y, axis=1)
    stats_ref[...] += jnp.stack([s, q], axis=0)


def _k4_body(y2_ref, scale_ref, shift_ref, out_ref):
    out_ref[0] = jnp.maximum(y2_ref[0] * scale_ref[...] + shift_ref[...], 0.0)


def _conv2_pass(y1, sc1, sh1, W2, b2, CM1, CM2, nb):
    BH = y1.shape[0]
    N = y1.shape[2]
    return pl.pallas_call(
        _k3_body,
        grid=(BH, nb),
        in_specs=[
            pl.BlockSpec((1, CM1, NT), lambda b, n: (b, 0, n)),
            pl.BlockSpec((CM1, 1), lambda b, n: (0, 0)),
            pl.BlockSpec((CM1, 1), lambda b, n: (0, 0)),
            pl.BlockSpec((CM2, CM1), lambda b, n: (0, 0)),
            pl.BlockSpec((CM2, 1), lambda b, n: (0, 0)),
        ],
        out_specs=[
            pl.BlockSpec((1, CM2, NT), lambda b, n: (b, 0, n)),
            pl.BlockSpec((2, CM2), lambda b, n: (0, 0)),
        ],
        out_shape=[
            jax.ShapeDtypeStruct((BH, CM2, N), jnp.float32),
            jax.ShapeDtypeStruct((2, CM2), jnp.float32),
        ],
        compiler_params=pltpu.CompilerParams(
            dimension_semantics=("arbitrary", "arbitrary")),
    )(y1, sc1, sh1, W2, b2)


def _bn_relu_pass(y2, sc2, sh2, CM2):
    BH = y2.shape[0]
    N = y2.shape[2]
    return pl.pallas_call(
        _k4_body,
        grid=(BH,),
        in_specs=[
            pl.BlockSpec((1, CM2, N), lambda b: (b, 0, 0)),
            pl.BlockSpec((CM2, 1), lambda b: (0, 0)),
            pl.BlockSpec((CM2, 1), lambda b: (0, 0)),
        ],
        out_specs=pl.BlockSpec((1, CM2, N), lambda b: (b, 0, 0)),
        out_shape=jax.ShapeDtypeStruct((BH, CM2, N), jnp.float32),
    )(y2, sc2, sh2)


def kernel(point_coordinates1, point_coordinates2, features1, features2,
           W1, b1, g1, be1, W2, b2, g2, be2):
    B, _, N = point_coordinates1.shape
    S = point_coordinates2.shape[-1]
    C1 = features1.shape[1]
    C2 = features2.shape[1]
    CM1 = W1.shape[0]
    CM2 = W2.shape[0]
    nb = N // NT
    Q = B * N
    BH = B // 2
    QH = BH * N
    qpw = QH // 32

    w1a = W1[:, :C1]
    w1b = W1[:, C1:]
    b1c = b1[:, None]

    gidx, wgt = pl.pallas_call(
        _k1a_body,
        grid=(BH, nb),
        in_specs=[
            pl.BlockSpec((1, 3, NT), lambda b, n: (b, 0, n)),
            pl.BlockSpec((1, 3, S), lambda b, n: (b, 0, 0)),
        ],
        out_specs=[
            pl.BlockSpec((1, 3, NT), lambda b, n: (b, 0, n)),
            pl.BlockSpec((1, 3, NT), lambda b, n: (b, 0, n)),
        ],
        out_shape=[
            jax.ShapeDtypeStruct((BH, 3, N), jnp.int32),
            jax.ShapeDtypeStruct((BH, 3, N), jnp.float32),
        ],
        compiler_params=pltpu.CompilerParams(
            dimension_semantics=("arbitrary", "arbitrary")),
    )(point_coordinates1[:BH], point_coordinates2[:BH])

    table = features2[:BH].transpose(0, 2, 1).reshape(BH * S, C2)

    mesh = plsc.VectorSubcoreMesh(core_axis_name="c", subcore_axis_name="s")
    interp = pl.kernel(
        _sc_interp_body,
        mesh=mesh,
        out_type=jax.ShapeDtypeStruct((QH, C2), jnp.float32),
        scratch_types=[
            pltpu.VMEM((qpw,), jnp.int32),
            pltpu.VMEM((qpw,), jnp.int32),
            pltpu.VMEM((qpw,), jnp.int32),
            pltpu.VMEM((CH, C2), jnp.float32),
            pltpu.VMEM((CH, C2), jnp.float32),
            pltpu.VMEM((CH, C2), jnp.float32),
            pltpu.VMEM((CH, C2), jnp.float32),
            pltpu.VMEM((CH, C2), jnp.float32),
            pltpu.VMEM((CH, C2), jnp.float32),
            pltpu.VMEM((CH, 16), jnp.float32),
            pltpu.VMEM((CH, 16), jnp.float32),
            pltpu.VMEM((CH, 16), jnp.float32),
            pltpu.VMEM((CH, 16), jnp.float32),
            pltpu.VMEM((CH, 16), jnp.float32),
            pltpu.VMEM((CH, 16), jnp.float32),
            pltpu.VMEM((CH, C2), jnp.float32),
            pltpu.SemaphoreType.DMA,
            pltpu.SemaphoreType.DMA,
        ],
    )(table,
      gidx[:, 0, :].reshape(QH), gidx[:, 1, :].reshape(QH),
      gidx[:, 2, :].reshape(QH),
      jnp.broadcast_to(wgt[:, 0, :].reshape(QH)[:, None], (QH, 16)),
      jnp.broadcast_to(wgt[:, 1, :].reshape(QH)[:, None], (QH, 16)),
      jnp.broadcast_to(wgt[:, 2, :].reshape(QH)[:, None], (QH, 16)))

    y1b, stats1b = pl.pallas_call(
        _k1b_body,
        grid=(BH, nb),
        in_specs=[
            pl.BlockSpec((1, 3, NT), lambda b, n: (b, 0, n)),
            pl.BlockSpec((1, 3, S), lambda b, n: (b, 0, 0)),
            pl.BlockSpec((1, C1, NT), lambda b, n: (b, 0, n)),
            pl.BlockSpec((1, C2, S), lambda b, n: (b, 0, 0)),
            pl.BlockSpec((CM1, C1), lambda b, n: (0, 0)),
            pl.BlockSpec((CM1, C2), lambda b, n: (0, 0)),
            pl.BlockSpec((CM1, 1), lambda b, n: (0, 0)),
        ],
        out_specs=[
            pl.BlockSpec((1, CM1, NT), lambda b, n: (b, 0, n)),
            pl.BlockSpec((2, CM1), lambda b, n: (0, 0)),
        ],
        out_shape=[
            jax.ShapeDtypeStruct((BH, CM1, N), jnp.float32),
            jax.ShapeDtypeStruct((2, CM1), jnp.float32),
        ],
        compiler_params=pltpu.CompilerParams(
            dimension_semantics=("arbitrary", "arbitrary")),
    )(point_coordinates1[BH:], point_coordinates2[BH:],
      features1[BH:], features2[BH:], w1a, w1b, b1c)

    y1a, stats1a = pl.pallas_call(
        _k2_body,
        grid=(BH, nb),
        in_specs=[
            pl.BlockSpec((1, C1, NT), lambda b, n: (b, 0, n)),
            pl.BlockSpec((1, NT, C2), lambda b, n: (b, n, 0)),
            pl.BlockSpec((CM1, C1), lambda b, n: (0, 0)),
            pl.BlockSpec((CM1, C2), lambda b, n: (0, 0)),
            pl.BlockSpec((CM1, 1), lambda b, n: (0, 0)),
        ],
        out_specs=[
            pl.BlockSpec((1, CM1, NT), lambda b, n: (b, 0, n)),
            pl.BlockSpec((2, CM1), lambda b, n: (0, 0)),
        ],
        out_shape=[
            jax.ShapeDtypeStruct((BH, CM1, N), jnp.float32),
            jax.ShapeDtypeStruct((2, CM1), jnp.float32),
        ],
        compiler_params=pltpu.CompilerParams(
            dimension_semantics=("arbitrary", "arbitrary")),
    )(features1[:BH], interp.reshape(BH, N, C2), w1a, w1b, b1c)

    stats1 = stats1a + stats1b
    mean1 = stats1[0] / Q
    var1 = stats1[1] / Q - mean1 * mean1
    sc1 = (g1 / jnp.sqrt(var1 + EPS_BN))[:, None]
    sh1 = (be1 - mean1 * sc1[:, 0])[:, None]

    y2a, stats2a = _conv2_pass(y1a, sc1, sh1, W2, b2[:, None], CM1, CM2, nb)
    y2b, stats2b = _conv2_pass(y1b, sc1, sh1, W2, b2[:, None], CM1, CM2, nb)

    stats2 = stats2a + stats2b
    mean2 = stats2[0] / Q
    var2 = stats2[1] / Q - mean2 * mean2
    sc2 = (g2 / jnp.sqrt(var2 + EPS_BN))[:, None]
    sh2 = (be2 - mean2 * sc2[:, 0])[:, None]

    outa = _bn_relu_pass(y2a, sc2, sh2, CM2)
    outb = _bn_relu_pass(y2b, sc2, sh2, CM2)
    return jnp.concatenate([outa, outb], axis=0)


# speedup vs baseline: 1.5146x; 1.0041x over previous
import functools

import jax
import jax.numpy as jnp
from jax import lax
from jax.experimental import pallas as pl
from jax.experimental.pallas import tpu as pltpu
from jax.experimental.pallas import tpu_sc as plsc

EPS_BN = 1e-5
NT = 512
CH = 16


def _top3(x1, x2, nt, S):
    m = jax.lax.dot_general(x1, x2, (((0,), (0,)), ((), ())),
                            preferred_element_type=jnp.float32)
    n1 = (x1[0] * x1[0] + x1[1] * x1[1]) + x1[2] * x1[2]
    n2 = (x2[0] * x2[0] + x2[1] * x2[1]) + x2[2] * x2[2]
    d = (-2.0 * m + n1[:, None]) + n2[None, :]

    iota = jax.lax.broadcasted_iota(jnp.int32, (nt, S), 1)
    idxs = []
    vals = []
    for _ in range(3):
        mn = jnp.min(d, axis=1)
        ii = jnp.min(jnp.where(d == mn[:, None], iota, S), axis=1)
        idxs.append(ii)
        vals.append(mn)
        d = jnp.where(iota == ii[:, None], jnp.float32(jnp.inf), d)

    r0 = 1.0 / (vals[0] + 1e-8)
    r1 = 1.0 / (vals[1] + 1e-8)
    r2 = 1.0 / (vals[2] + 1e-8)
    rs = r0 + r1 + r2
    return idxs, (r0 / rs, r1 / rs, r2 / rs)


def _k1a_body(xyz1_ref, xyz2_ref, idx_ref, w_ref):
    S = xyz2_ref.shape[-1]
    nt = xyz1_ref.shape[-1]
    idxs, ws = _top3(xyz1_ref[0], xyz2_ref[0], nt, S)
    base = pl.program_id(0) * S
    idx_ref[0] = jnp.stack([idxs[0] + base, idxs[1] + base, idxs[2] + base])
    w_ref[0] = jnp.stack(list(ws))


def _k1b_body(xyz1_ref, xyz2_ref, f1_ref, f2_ref, w1a_ref, w1b_ref, b1_ref,
              y1_ref, stats_ref):
    S = xyz2_ref.shape[-1]
    nt = xyz1_ref.shape[-1]
    idxs, ws = _top3(xyz1_ref[0], xyz2_ref[0], nt, S)

    iota_s = jax.lax.broadcasted_iota(jnp.int32, (S, nt), 0)
    at = jnp.where(iota_s == idxs[0][None, :], ws[0][None, :], 0.0)
    at = at + jnp.where(iota_s == idxs[1][None, :], ws[1][None, :], 0.0)
    at = at + jnp.where(iota_s == idxs[2][None, :], ws[2][None, :], 0.0)

    interp = jax.lax.dot_general(f2_ref[0], at, (((1,), (0,)), ((), ())),
                                 preferred_element_type=jnp.float32,
                                 precision=jax.lax.Precision.HIGHEST)

    y = jax.lax.dot_general(w1a_ref[...], f1_ref[0], (((1,), (0,)), ((), ())),
                            preferred_element_type=jnp.float32)
    y = y + jax.lax.dot_general(w1b_ref[...], interp, (((1,), (0,)), ((), ())),
                                preferred_element_type=jnp.float32)
    y = y + b1_ref[...]
    y1_ref[0] = y

    b = pl.program_id(0)
    n = pl.program_id(1)

    @pl.when(jnp.logical_and(b == 0, n == 0))
    def _():
        stats_ref[...] = jnp.zeros_like(stats_ref)

    s = jnp.sum(y, axis=1)
    q = jnp.sum(y * y, axis=1)
    stats_ref[...] += jnp.stack([s, q], axis=0)


def _sc_interp_body(table, i0, i1, i2, w0e, w1e, w2e, out,
                    iv0, iv1, iv2,
                    r0a, r0b, r1a, r1b, r2a, r2b,
                    wb0a, wb0b, wb1a, wb1b, wb2a, wb2b, ob,
                    sg0, sg1):
    Q = out.shape[0]
    C2 = out.shape[1]
    NW = 32
    qpw = Q // NW
    nchunks = qpw // CH
    wid = lax.axis_index("s") * 2 + lax.axis_index("c")
    base = pl.multiple_of(wid * qpw, 8)

    pltpu.sync_copy(i0.at[pl.ds(base, qpw)], iv0)
    pltpu.sync_copy(i1.at[pl.ds(base, qpw)], iv1)
    pltpu.sync_copy(i2.at[pl.ds(base, qpw)], iv2)

    rbufs = ((r0a, r1a, r2a), (r0b, r1b, r2b))
    wbufs = ((wb0a, wb1a, wb2a), (wb0b, wb1b, wb2b))
    ivs = (iv0, iv1, iv2)
    wes = (w0e, w1e, w2e)
    sgs = (sg0, sg1)

    def issue(g, buf):
        off = pl.multiple_of(g * CH, 8)
        qs = pl.multiple_of(base + g * CH, 8)
        for k in range(3):
            pltpu.async_copy(table.at[ivs[k].at[pl.ds(off, CH)]],
                             rbufs[buf][k], sgs[buf])
            pltpu.async_copy(wes[k].at[pl.ds(qs, CH)], wbufs[buf][k], sgs[buf])

    def drain(g, buf):
        off = pl.multiple_of(g * CH, 8)
        qs = pl.multiple_of(base + g * CH, 8)
        for k in range(3):
            pltpu.make_async_copy(table.at[ivs[k].at[pl.ds(off, CH)]],
                                  rbufs[buf][k], sgs[buf]).wait()
            pltpu.make_async_copy(wes[k].at[pl.ds(qs, CH)],
                                  wbufs[buf][k], sgs[buf]).wait()

    issue(0, 0)
    issue(1, 1)

    def step(g, buf):
        drain(g, buf)
        rb0, rb1, rb2 = rbufs[buf]
        wb0, wb1, wb2 = wbufs[buf]
        for q in range(CH):
            a0 = wb0[q]
            a1 = wb1[q]
            a2 = wb2[q]
            for c in range(C2 // 16):
                sl = pl.ds(c * 16, 16)
                ob[q, sl] = (a0 * rb0[q, sl] + a1 * rb1[q, sl]) + a2 * rb2[q, sl]

        @pl.when(g + 2 < nchunks)
        def _():
            issue(g + 2, buf)

        pltpu.sync_copy(ob, out.at[pl.ds(base + g * CH, CH)])

    def body(gg, carry):
        step(gg * 2, 0)
        step(gg * 2 + 1, 1)
        return carry

    lax.fori_loop(0, nchunks // 2, body, 0)


def _k2_body(f1_ref, interp_ref, w1a_ref, w1b_ref, b1_ref, y1_ref, stats_ref):
    y = jax.lax.dot_general(w1a_ref[...], f1_ref[0], (((1,), (0,)), ((), ())),
                            preferred_element_type=jnp.float32)
    y = y + jax.lax.dot_general(w1b_ref[...], interp_ref[0],
                                (((1,), (1,)), ((), ())),
                                preferred_element_type=jnp.float32)
    y = y + b1_ref[...]
    y1_ref[0] = y

    b = pl.program_id(0)
    n = pl.program_id(1)

    @pl.when(jnp.logical_and(b == 0, n == 0))
    def _():
        stats_ref[...] = jnp.zeros_like(stats_ref)

    s = jnp.sum(y, axis=1)
    q = jnp.sum(y * y, axis=1)
    stats_ref[...] += jnp.stack([s, q], axis=0)


def _k3_body(y1_ref, scale_ref, shift_ref, w2_ref, b2_ref, y2_ref, stats_ref):
    x = y1_ref[0]
    x = jnp.maximum(x * scale_ref[...] + shift_ref[...], 0.0)
    y = jax.lax.dot_general(w2_ref[...], x, (((1,), (0,)), ((), ())),
                            preferred_element_type=jnp.float32)
    y = y + b2_ref[...]
    y2_ref[0] = y

    b = pl.program_id(0)
    n = pl.program_id(1)

    @pl.when(jnp.logical_and(b == 0, n == 0))
    def _():
        stats_ref[...] = jnp.zeros_like(stats_ref)

    s = jnp.sum(y, axis=1)
    q = jnp.sum(y * y, axis=1)
    stats_ref[...] += jnp.stack([s, q], axis=0)


def _k4_body(y2_ref, scale_ref, shift_ref, out_ref):
    out_ref[0] = jnp.maximum(y2_ref[0] * scale_ref[...] + shift_ref[...], 0.0)


def _conv2_pass(y1, sc1, sh1, W2, b2, CM1, CM2, nb):
    BH = y1.shape[0]
    N = y1.shape[2]
    return pl.pallas_call(
        _k3_body,
        grid=(BH, nb),
        in_specs=[
            pl.BlockSpec((1, CM1, NT), lambda b, n: (b, 0, n)),
            pl.BlockSpec((CM1, 1), lambda b, n: (0, 0)),
            pl.BlockSpec((CM1, 1), lambda b, n: (0, 0)),
            pl.BlockSpec((CM2, CM1), lambda b, n: (0, 0)),
            pl.BlockSpec((CM2, 1), lambda b, n: (0, 0)),
        ],
        out_specs=[
            pl.BlockSpec((1, CM2, NT), lambda b, n: (b, 0, n)),
            pl.BlockSpec((2, CM2), lambda b, n: (0, 0)),
        ],
        out_shape=[
            jax.ShapeDtypeStruct((BH, CM2, N), jnp.float32),
            jax.ShapeDtypeStruct((2, CM2), jnp.float32),
        ],
        compiler_params=pltpu.CompilerParams(
            dimension_semantics=("arbitrary", "arbitrary")),
    )(y1, sc1, sh1, W2, b2)


def _bn_relu_pass(y2, sc2, sh2, CM2):
    BH = y2.shape[0]
    N = y2.shape[2]
    return pl.pallas_call(
        _k4_body,
        grid=(BH,),
        in_specs=[
            pl.BlockSpec((1, CM2, N), lambda b: (b, 0, 0)),
            pl.BlockSpec((CM2, 1), lambda b: (0, 0)),
            pl.BlockSpec((CM2, 1), lambda b: (0, 0)),
        ],
        out_specs=pl.BlockSpec((1, CM2, N), lambda b: (b, 0, 0)),
        out_shape=jax.ShapeDtypeStruct((BH, CM2, N), jnp.float32),
    )(y2, sc2, sh2)


def kernel(point_coordinates1, point_coordinates2, features1, features2,
           W1, b1, g1, be1, W2, b2, g2, be2):
    B, _, N = point_coordinates1.shape
    S = point_coordinates2.shape[-1]
    C1 = features1.shape[1]
    C2 = features2.shape[1]
    CM1 = W1.shape[0]
    CM2 = W2.shape[0]
    nb = N // NT
    Q = B * N
    BS = 10
    BT = B - BS
    QH = BS * N
    qpw = QH // 32

    w1a = W1[:, :C1]
    w1b = W1[:, C1:]
    b1c = b1[:, None]

    gidx, wgt = pl.pallas_call(
        _k1a_body,
        grid=(BS, nb),
        in_specs=[
            pl.BlockSpec((1, 3, NT), lambda b, n: (b, 0, n)),
            pl.BlockSpec((1, 3, S), lambda b, n: (b, 0, 0)),
        ],
        out_specs=[
            pl.BlockSpec((1, 3, NT), lambda b, n: (b, 0, n)),
            pl.BlockSpec((1, 3, NT), lambda b, n: (b, 0, n)),
        ],
        out_shape=[
            jax.ShapeDtypeStruct((BS, 3, N), jnp.int32),
            jax.ShapeDtypeStruct((BS, 3, N), jnp.float32),
        ],
        compiler_params=pltpu.CompilerParams(
            dimension_semantics=("arbitrary", "arbitrary")),
    )(point_coordinates1[:BS], point_coordinates2[:BS])

    table = features2[:BS].transpose(0, 2, 1).reshape(BS * S, C2)

    mesh = plsc.VectorSubcoreMesh(core_axis_name="c", subcore_axis_name="s")
    interp = pl.kernel(
        _sc_interp_body,
        mesh=mesh,
        out_type=jax.ShapeDtypeStruct((QH, C2), jnp.float32),
        scratch_types=[
            pltpu.VMEM((qpw,), jnp.int32),
            pltpu.VMEM((qpw,), jnp.int32),
            pltpu.VMEM((qpw,), jnp.int32),
            pltpu.VMEM((CH, C2), jnp.float32),
            pltpu.VMEM((CH, C2), jnp.float32),
            pltpu.VMEM((CH, C2), jnp.float32),
            pltpu.VMEM((CH, C2), jnp.float32),
            pltpu.VMEM((CH, C2), jnp.float32),
            pltpu.VMEM((CH, C2), jnp.float32),
            pltpu.VMEM((CH, 16), jnp.float32),
            pltpu.VMEM((CH, 16), jnp.float32),
            pltpu.VMEM((CH, 16), jnp.float32),
            pltpu.VMEM((CH, 16), jnp.float32),
            pltpu.VMEM((CH, 16), jnp.float32),
            pltpu.VMEM((CH, 16), jnp.float32),
            pltpu.VMEM((CH, C2), jnp.float32),
            pltpu.SemaphoreType.DMA,
            pltpu.SemaphoreType.DMA,
        ],
    )(table,
      gidx[:, 0, :].reshape(QH), gidx[:, 1, :].reshape(QH),
      gidx[:, 2, :].reshape(QH),
      jnp.broadcast_to(wgt[:, 0, :].reshape(QH)[:, None], (QH, 16)),
      jnp.broadcast_to(wgt[:, 1, :].reshape(QH)[:, None], (QH, 16)),
      jnp.broadcast_to(wgt[:, 2, :].reshape(QH)[:, None], (QH, 16)))

    y1b, stats1b = pl.pallas_call(
        _k1b_body,
        grid=(BT, nb),
        in_specs=[
            pl.BlockSpec((1, 3, NT), lambda b, n: (b, 0, n)),
            pl.BlockSpec((1, 3, S), lambda b, n: (b, 0, 0)),
            pl.BlockSpec((1, C1, NT), lambda b, n: (b, 0, n)),
            pl.BlockSpec((1, C2, S), lambda b, n: (b, 0, 0)),
            pl.BlockSpec((CM1, C1), lambda b, n: (0, 0)),
            pl.BlockSpec((CM1, C2), lambda b, n: (0, 0)),
            pl.BlockSpec((CM1, 1), lambda b, n: (0, 0)),
        ],
        out_specs=[
            pl.BlockSpec((1, CM1, NT), lambda b, n: (b, 0, n)),
            pl.BlockSpec((2, CM1), lambda b, n: (0, 0)),
        ],
        out_shape=[
            jax.ShapeDtypeStruct((BT, CM1, N), jnp.float32),
            jax.ShapeDtypeStruct((2, CM1), jnp.float32),
        ],
        compiler_params=pltpu.CompilerParams(
            dimension_semantics=("arbitrary", "arbitrary")),
    )(point_coordinates1[BS:], point_coordinates2[BS:],
      features1[BS:], features2[BS:], w1a, w1b, b1c)

    y1a, stats1a = pl.pallas_call(
        _k2_body,
        grid=(BS, nb),
        in_specs=[
            pl.BlockSpec((1, C1, NT), lambda b, n: (b, 0, n)),
            pl.BlockSpec((1, NT, C2), lambda b, n: (b, n, 0)),
            pl.BlockSpec((CM1, C1), lambda b, n: (0, 0)),
            pl.BlockSpec((CM1, C2), lambda b, n: (0, 0)),
            pl.BlockSpec((CM1, 1), lambda b, n: (0, 0)),
        ],
        out_specs=[
            pl.BlockSpec((1, CM1, NT), lambda b, n: (b, 0, n)),
            pl.BlockSpec((2, CM1), lambda b, n: (0, 0)),
        ],
        out_shape=[
            jax.ShapeDtypeStruct((BS, CM1, N), jnp.float32),
            jax.ShapeDtypeStruct((2, CM1), jnp.float32),
        ],
        compiler_params=pltpu.CompilerParams(
            dimension_semantics=("arbitrary", "arbitrary")),
    )(features1[:BS], interp.reshape(BS, N, C2), w1a, w1b, b1c)

    stats1 = stats1a + stats1b
    mean1 = stats1[0] / Q
    var1 = stats1[1] / Q - mean1 * mean1
    sc1 = (g1 / jnp.sqrt(var1 + EPS_BN))[:, None]
    sh1 = (be1 - mean1 * sc1[:, 0])[:, None]

    y2a, stats2a = _conv2_pass(y1a, sc1, sh1, W2, b2[:, None], CM1, CM2, nb)
    y2b, stats2b = _conv2_pass(y1b, sc1, sh1, W2, b2[:, None], CM1, CM2, nb)

    stats2 = stats2a + stats2b
    mean2 = stats2[0] / Q
    var2 = stats2[1] / Q - mean2 * mean2
    sc2 = (g2 / jnp.sqrt(var2 + EPS_BN))[:, None]
    sh2 = (be2 - mean2 * sc2[:, 0])[:, None]

    outa = _bn_relu_pass(y2a, sc2, sh2, CM2)
    outb = _bn_relu_pass(y2b, sc2, sh2, CM2)
    return jnp.concatenate([outa, outb], axis=0)
